# Initial kernel scaffold; baseline (speedup 1.0000x reference)
import functools

import jax
import jax.numpy as jnp
from jax import lax
from jax.experimental import pallas as pl
from jax.experimental.pallas import tpu as pltpu
from jax.experimental.pallas import tpu_sc as plsc

B, S, D = 4, 2048, 1024
N = 512
WD = 128
OUT_D = 2 * D + WD
G = B * N
NC, NS, L = 2, 16, 16
NW = NC * NS
SPW = G // NW
CH = 32
NCHUNK = SPW // CH

_mesh = plsc.VectorSubcoreMesh(core_axis_name="c", subcore_axis_name="s")


@functools.partial(
    pl.kernel,
    mesh=_mesh,
    out_type=jax.ShapeDtypeStruct((G, OUT_D), jnp.float32),
    scratch_types=[
        pltpu.VMEM((SPW * 2,), jnp.int32),
        pltpu.VMEM((CH,), jnp.int32),
        pltpu.VMEM((CH,), jnp.int32),
        pltpu.VMEM((CH,), jnp.int32),
        pltpu.VMEM((CH, D), jnp.float32),
        pltpu.VMEM((CH, D), jnp.float32),
        pltpu.VMEM((CH, WD), jnp.float32),
        pltpu.SemaphoreType.DMA,
    ],
)
def _span_extract(seq_hbm, spans_hbm, wtab_hbm, out_hbm,
                  spans_v, sidx_v, eidx_v, widx_v,
                  srow_v, erow_v, wrow_v, sem):
    wid = lax.axis_index("s") * NC + lax.axis_index("c")
    base = wid * SPW
    boff = (base // N) * S
    pltpu.sync_copy(spans_hbm.at[pl.ds(base * 2, SPW * 2)], spans_v)
    for c in range(NCHUNK):
        cbase = base + c * CH
        for j in range(CH // L):
            lane = lax.iota(jnp.int32, L)
            pos = 2 * (c * CH + j * L) + 2 * lane
            starts = plsc.load_gather(spans_v, [pos])
            ends = plsc.load_gather(spans_v, [pos + 1])
            sidx_v[pl.ds(j * L, L)] = starts + boff
            eidx_v[pl.ds(j * L, L)] = ends + boff
            widx_v[pl.ds(j * L, L)] = ends - starts
        h1 = pltpu.async_copy(seq_hbm.at[sidx_v], srow_v, sem)
        h2 = pltpu.async_copy(seq_hbm.at[eidx_v], erow_v, sem)
        h3 = pltpu.async_copy(wtab_hbm.at[widx_v], wrow_v, sem)
        h1.wait()
        h2.wait()
        h3.wait()
        pltpu.sync_copy(srow_v, out_hbm.at[pl.ds(cbase, CH), pl.ds(0, D)])
        pltpu.sync_copy(erow_v, out_hbm.at[pl.ds(cbase, CH), pl.ds(D, D)])
        pltpu.sync_copy(wrow_v, out_hbm.at[pl.ds(cbase, CH), pl.ds(2 * D, WD)])


def kernel(sequence_tensor, span_indices, width_table):
    seq_flat = sequence_tensor.reshape(B * S, D)
    spans_flat = span_indices.astype(jnp.int32).reshape(G * 2)
    out = _span_extract(seq_flat, spans_flat, width_table)
    return out.reshape(B, N, OUT_D)


# speedup vs baseline: 1.6890x; 1.6890x over previous
import functools

import jax
import jax.numpy as jnp
from jax import lax
from jax.experimental import pallas as pl
from jax.experimental.pallas import tpu as pltpu
from jax.experimental.pallas import tpu_sc as plsc

B, S, D = 4, 2048, 1024
N = 512
WD = 128
OUT_D = 2 * D + WD
G = B * N
NC, NS, L = 2, 16, 16
NW = NC * NS
SPW = G // NW
CH = 32
NCHUNK = SPW // CH

_mesh = plsc.VectorSubcoreMesh(core_axis_name="c", subcore_axis_name="s")


@functools.partial(
    pl.kernel,
    mesh=_mesh,
    out_type=jax.ShapeDtypeStruct((G, OUT_D), jnp.float32),
    scratch_types=[
        pltpu.VMEM((CH,), jnp.int32),
        pltpu.VMEM((CH,), jnp.int32),
        pltpu.VMEM((CH,), jnp.int32),
        pltpu.VMEM((CH,), jnp.int32),
        pltpu.VMEM((CH,), jnp.int32),
        pltpu.VMEM((CH, D), jnp.float32),
        pltpu.VMEM((CH, D), jnp.float32),
        pltpu.VMEM((CH, WD), jnp.float32),
        pltpu.SemaphoreType.DMA,
    ],
)
def _span_extract(seq_hbm, starts_hbm, ends_hbm, wtab_hbm, out_hbm,
                  sv_v, ev_v, sidx_v, eidx_v, widx_v,
                  srow_v, erow_v, wrow_v, sem):
    wid = lax.axis_index("s") * NC + lax.axis_index("c")
    base = wid * SPW
    boff = (base // N) * S
    for c in range(NCHUNK):
        cbase = base + c * CH
        pltpu.sync_copy(starts_hbm.at[pl.ds(cbase, CH)], sv_v)
        pltpu.sync_copy(ends_hbm.at[pl.ds(cbase, CH)], ev_v)
        for j in range(CH // L):
            s = sv_v[pl.ds(j * L, L)]
            e = ev_v[pl.ds(j * L, L)]
            sidx_v[pl.ds(j * L, L)] = s + boff
            eidx_v[pl.ds(j * L, L)] = e + boff
            widx_v[pl.ds(j * L, L)] = e - s
        h1 = pltpu.async_copy(seq_hbm.at[sidx_v], srow_v, sem)
        h2 = pltpu.async_copy(seq_hbm.at[eidx_v], erow_v, sem)
        h3 = pltpu.async_copy(wtab_hbm.at[widx_v], wrow_v, sem)
        h1.wait()
        h2.wait()
        h3.wait()
        pltpu.sync_copy(srow_v, out_hbm.at[pl.ds(cbase, CH), pl.ds(0, D)])
        pltpu.sync_copy(erow_v, out_hbm.at[pl.ds(cbase, CH), pl.ds(D, D)])
        pltpu.sync_copy(wrow_v, out_hbm.at[pl.ds(cbase, CH), pl.ds(2 * D, WD)])


def kernel(sequence_tensor, span_indices, width_table):
    seq_flat = sequence_tensor.reshape(B * S, D)
    spans = span_indices.astype(jnp.int32)
    starts_flat = spans[..., 0].reshape(G)
    ends_flat = spans[..., 1].reshape(G)
    out = _span_extract(seq_flat, starts_flat, ends_flat, width_table)
    return out.reshape(B, N, OUT_D)


# speedup vs baseline: 1.6971x; 1.0048x over previous
import functools

import jax
import jax.numpy as jnp
from jax import lax
from jax.experimental import pallas as pl
from jax.experimental.pallas import tpu as pltpu
from jax.experimental.pallas import tpu_sc as plsc

B, S, D = 4, 2048, 1024
N = 512
WD = 128
OUT_D = 2 * D + WD
G = B * N
NC, NS, L = 2, 16, 16
NW = NC * NS
SPW = G // NW
CH = 32
NCHUNK = SPW // CH

_mesh = plsc.VectorSubcoreMesh(core_axis_name="c", subcore_axis_name="s")


@functools.partial(
    pl.kernel,
    mesh=_mesh,
    out_type=jax.ShapeDtypeStruct((G, OUT_D), jnp.float32),
    scratch_types=[
        pltpu.VMEM((CH,), jnp.int32),
        pltpu.VMEM((CH,), jnp.int32),
        pltpu.VMEM((CH,), jnp.int32),
        pltpu.VMEM((CH,), jnp.int32),
        pltpu.VMEM((CH,), jnp.int32),
        pltpu.VMEM((CH, OUT_D), jnp.float32),
        pltpu.SemaphoreType.DMA,
    ],
)
def _span_extract(seq_hbm, starts_hbm, ends_hbm, wtab_hbm, out_hbm,
                  sv_v, ev_v, sidx_v, eidx_v, widx_v,
                  orow_v, sem):
    wid = lax.axis_index("s") * NC + lax.axis_index("c")
    base = wid * SPW
    boff = (base // N) * S
    for c in range(NCHUNK):
        cbase = base + c * CH
        pltpu.sync_copy(starts_hbm.at[pl.ds(cbase, CH)], sv_v)
        pltpu.sync_copy(ends_hbm.at[pl.ds(cbase, CH)], ev_v)
        for j in range(CH // L):
            s = sv_v[pl.ds(j * L, L)]
            e = ev_v[pl.ds(j * L, L)]
            sidx_v[pl.ds(j * L, L)] = s + boff
            eidx_v[pl.ds(j * L, L)] = e + boff
            widx_v[pl.ds(j * L, L)] = e - s
        h1 = pltpu.async_copy(seq_hbm.at[sidx_v], orow_v.at[:, pl.ds(0, D)], sem)
        h2 = pltpu.async_copy(seq_hbm.at[eidx_v], orow_v.at[:, pl.ds(D, D)], sem)
        h3 = pltpu.async_copy(wtab_hbm.at[widx_v], orow_v.at[:, pl.ds(2 * D, WD)], sem)
        h1.wait()
        h2.wait()
        h3.wait()
        pltpu.sync_copy(orow_v, out_hbm.at[pl.ds(cbase, CH)])


def kernel(sequence_tensor, span_indices, width_table):
    seq_flat = sequence_tensor.reshape(B * S, D)
    spans = span_indices.astype(jnp.int32)
    starts_flat = spans[..., 0].reshape(G)
    ends_flat = spans[..., 1].reshape(G)
    out = _span_extract(seq_flat, starts_flat, ends_flat, width_table)
    return out.reshape(B, N, OUT_D)


# speedup vs baseline: 1.7073x; 1.0060x over previous
import functools

import jax
import jax.numpy as jnp
from jax import lax
from jax.experimental import pallas as pl
from jax.experimental.pallas import tpu as pltpu
from jax.experimental.pallas import tpu_sc as plsc

B, S, D = 4, 2048, 1024
N = 512
WD = 128
OUT_D = 2 * D + WD
G = B * N
NC, NS, L = 2, 16, 16
NW = NC * NS
SPW = G // NW
CH = 16
NCHUNK = SPW // CH
K = 2

_mesh = plsc.VectorSubcoreMesh(core_axis_name="c", subcore_axis_name="s")


@functools.partial(
    pl.kernel,
    mesh=_mesh,
    out_type=jax.ShapeDtypeStruct((G, OUT_D), jnp.float32),
    scratch_types=(
        [
            pltpu.VMEM((NCHUNK, CH), jnp.int32),
            pltpu.VMEM((NCHUNK, CH), jnp.int32),
            pltpu.VMEM((NCHUNK, CH), jnp.int32),
            pltpu.VMEM((NCHUNK, CH), jnp.int32),
            pltpu.VMEM((NCHUNK, CH), jnp.int32),
        ]
        + [pltpu.VMEM((CH, OUT_D), jnp.float32) for _ in range(K)]
        + [pltpu.SemaphoreType.DMA for _ in range(2 * K)]
    ),
)
def _span_extract(seq_hbm, starts_hbm, ends_hbm, wtab_hbm, out_hbm,
                  sv_v, ev_v, sidx_v, eidx_v, widx_v, *bufs):
    orow = bufs[:K]
    gsem = bufs[K:2 * K]
    wsem = bufs[2 * K:3 * K]
    wid = lax.axis_index("s") * NC + lax.axis_index("c")
    base = wid * SPW
    boff = (base // N) * S
    pltpu.sync_copy(starts_hbm.at[pl.ds(wid * NCHUNK, NCHUNK)], sv_v)
    pltpu.sync_copy(ends_hbm.at[pl.ds(wid * NCHUNK, NCHUNK)], ev_v)
    for c in range(NCHUNK):
        s = sv_v[c]
        e = ev_v[c]
        sidx_v[c] = s + boff
        eidx_v[c] = e + boff
        widx_v[c] = e - s

    gh = [None] * NCHUNK
    wh = [None] * K

    def fire(c):
        k = c % K
        if wh[k] is not None:
            wh[k].wait()
            wh[k] = None
        gh[c] = (
            pltpu.async_copy(seq_hbm.at[sidx_v.at[c]],
                             orow[k].at[:, pl.ds(0, D)], gsem[k]),
            pltpu.async_copy(seq_hbm.at[eidx_v.at[c]],
                             orow[k].at[:, pl.ds(D, D)], gsem[k]),
            pltpu.async_copy(wtab_hbm.at[widx_v.at[c]],
                             orow[k].at[:, pl.ds(2 * D, WD)], gsem[k]),
        )

    for c in range(min(K, NCHUNK)):
        fire(c)
    for c in range(NCHUNK):
        k = c % K
        for h in gh[c]:
            h.wait()
        wh[k] = pltpu.async_copy(
            orow[k], out_hbm.at[pl.ds(base + c * CH, CH)], wsem[k])
        if c + K < NCHUNK:
            fire(c + K)
    for k in range(K):
        if wh[k] is not None:
            wh[k].wait()


def kernel(sequence_tensor, span_indices, width_table):
    seq_flat = sequence_tensor.reshape(B * S, D)
    spans = span_indices.astype(jnp.int32)
    starts_flat = spans[..., 0].reshape(G // CH, CH)
    ends_flat = spans[..., 1].reshape(G // CH, CH)
    out = _span_extract(seq_flat, starts_flat, ends_flat, width_table)
    return out.reshape(B, N, OUT_D)


# speedup vs baseline: 1.7533x; 1.0269x over previous
import functools

import jax
import jax.numpy as jnp
from jax import lax
from jax.experimental import pallas as pl
from jax.experimental.pallas import tpu as pltpu
from jax.experimental.pallas import tpu_sc as plsc

B, S, D = 4, 2048, 1024
N = 512
WD = 128
OUT_D = 2 * D + WD
G = B * N
NC, NS, L = 2, 16, 16
NW = NC * NS
SPW = G // NW
CH = 16
NCHUNK = SPW // CH
K = 3

_mesh = plsc.VectorSubcoreMesh(core_axis_name="c", subcore_axis_name="s")


@functools.partial(
    pl.kernel,
    mesh=_mesh,
    out_type=jax.ShapeDtypeStruct((G, OUT_D), jnp.float32),
    scratch_types=(
        [
            pltpu.VMEM((NCHUNK, CH), jnp.int32),
            pltpu.VMEM((NCHUNK, CH), jnp.int32),
            pltpu.VMEM((NCHUNK, CH), jnp.int32),
            pltpu.VMEM((NCHUNK, CH), jnp.int32),
            pltpu.VMEM((NCHUNK, CH), jnp.int32),
        ]
        + [pltpu.VMEM((CH, OUT_D), jnp.float32) for _ in range(K)]
        + [pltpu.SemaphoreType.DMA for _ in range(2 * K)]
    ),
)
def _span_extract(seq_hbm, starts_hbm, ends_hbm, wtab_hbm, out_hbm,
                  sv_v, ev_v, sidx_v, eidx_v, widx_v, *bufs):
    orow = bufs[:K]
    gsem = bufs[K:2 * K]
    wsem = bufs[2 * K:3 * K]
    wid = lax.axis_index("s") * NC + lax.axis_index("c")
    base = wid * SPW
    boff = (base // N) * S
    pltpu.sync_copy(starts_hbm.at[pl.ds(wid * NCHUNK, NCHUNK)], sv_v)
    pltpu.sync_copy(ends_hbm.at[pl.ds(wid * NCHUNK, NCHUNK)], ev_v)
    for c in range(NCHUNK):
        s = sv_v[c]
        e = ev_v[c]
        sidx_v[c] = s + boff
        eidx_v[c] = e + boff
        widx_v[c] = e - s

    gh = [None] * NCHUNK
    wh = [None] * K

    def fire(c):
        k = c % K
        if wh[k] is not None:
            wh[k].wait()
            wh[k] = None
        gh[c] = (
            pltpu.async_copy(seq_hbm.at[sidx_v.at[c]],
                             orow[k].at[:, pl.ds(0, D)], gsem[k]),
            pltpu.async_copy(seq_hbm.at[eidx_v.at[c]],
                             orow[k].at[:, pl.ds(D, D)], gsem[k]),
            pltpu.async_copy(wtab_hbm.at[widx_v.at[c]],
                             orow[k].at[:, pl.ds(2 * D, WD)], gsem[k]),
        )

    for c in range(min(K, NCHUNK)):
        fire(c)
    for c in range(NCHUNK):
        k = c % K
        for h in gh[c]:
            h.wait()
        wh[k] = pltpu.async_copy(
            orow[k], out_hbm.at[pl.ds(base + c * CH, CH)], wsem[k])
        if c + K < NCHUNK:
            fire(c + K)
    for k in range(K):
        if wh[k] is not None:
            wh[k].wait()


def kernel(sequence_tensor, span_indices, width_table):
    seq_flat = sequence_tensor.reshape(B * S, D)
    spans = span_indices.astype(jnp.int32)
    starts_flat = spans[..., 0].reshape(G // CH, CH)
    ends_flat = spans[..., 1].reshape(G // CH, CH)
    out = _span_extract(seq_flat, starts_flat, ends_flat, width_table)
    return out.reshape(B, N, OUT_D)


# speedup vs baseline: 1.7868x; 1.0191x over previous
import functools

import jax
import jax.numpy as jnp
from jax import lax
from jax.experimental import pallas as pl
from jax.experimental.pallas import tpu as pltpu
from jax.experimental.pallas import tpu_sc as plsc

B, S, D = 4, 2048, 1024
N = 512
WD = 128
OUT_D = 2 * D + WD
G = B * N
NC, NS, L = 2, 16, 16
NW = NC * NS
SPW = G // NW
CH = 16
NCHUNK = SPW // CH
K = 3

_mesh = plsc.VectorSubcoreMesh(core_axis_name="c", subcore_axis_name="s")


@functools.partial(
    pl.kernel,
    mesh=_mesh,
    out_type=jax.ShapeDtypeStruct((G, OUT_D), jnp.float32),
    scratch_types=(
        [
            pltpu.VMEM((SPW * 2,), jnp.int32),
            pltpu.VMEM((NCHUNK, CH), jnp.int32),
            pltpu.VMEM((NCHUNK, CH), jnp.int32),
            pltpu.VMEM((NCHUNK, CH), jnp.int32),
        ]
        + [pltpu.VMEM((CH, OUT_D), jnp.float32) for _ in range(K)]
        + [pltpu.SemaphoreType.DMA for _ in range(2 * K)]
    ),
)
def _span_extract(seq_hbm, spans_hbm, wtab_hbm, out_hbm,
                  spans_v, sidx_v, eidx_v, widx_v, *bufs):
    orow = bufs[:K]
    gsem = bufs[K:2 * K]
    wsem = bufs[2 * K:3 * K]
    wid = lax.axis_index("s") * NC + lax.axis_index("c")
    base = wid * SPW
    boff = (base // N) * S
    pltpu.sync_copy(spans_hbm.at[pl.ds(base * 2, SPW * 2)], spans_v)
    lane = lax.iota(jnp.int32, L)
    duo = (2 * lane) & (L - 1)
    half = lane < (L // 2)

    def _pick(v, idx):
        return lax.gather(
            v, idx[:, None],
            dimension_numbers=lax.GatherDimensionNumbers(
                offset_dims=(), collapsed_slice_dims=(0,),
                start_index_map=(0,)),
            slice_sizes=(1,),
            mode=lax.GatherScatterMode.PROMISE_IN_BOUNDS)

    for c in range(NCHUNK):
        a = spans_v[pl.ds(c * 2 * L, L)]
        b = spans_v[pl.ds(c * 2 * L + L, L)]
        s = jnp.where(half, _pick(a, duo), _pick(b, duo))
        e = jnp.where(half, _pick(a, duo + 1), _pick(b, duo + 1))
        sidx_v[c] = s + boff
        eidx_v[c] = e + boff
        widx_v[c] = e - s

    gh = [None] * NCHUNK
    wh = [None] * K

    def fire(c):
        k = c % K
        if wh[k] is not None:
            wh[k].wait()
            wh[k] = None
        gh[c] = (
            pltpu.async_copy(seq_hbm.at[sidx_v.at[c]],
                             orow[k].at[:, pl.ds(0, D)], gsem[k]),
            pltpu.async_copy(seq_hbm.at[eidx_v.at[c]],
                             orow[k].at[:, pl.ds(D, D)], gsem[k]),
            pltpu.async_copy(wtab_hbm.at[widx_v.at[c]],
                             orow[k].at[:, pl.ds(2 * D, WD)], gsem[k]),
        )

    for c in range(min(K, NCHUNK)):
        fire(c)
    for c in range(NCHUNK):
        k = c % K
        for h in gh[c]:
            h.wait()
        wh[k] = pltpu.async_copy(
            orow[k], out_hbm.at[pl.ds(base + c * CH, CH)], wsem[k])
        if c + K < NCHUNK:
            fire(c + K)
    for k in range(K):
        if wh[k] is not None:
            wh[k].wait()


def kernel(sequence_tensor, span_indices, width_table):
    seq_flat = sequence_tensor.reshape(B * S, D)
    spans_flat = span_indices.astype(jnp.int32).reshape(G * 2)
    out = _span_extract(seq_flat, spans_flat, width_table)
    return out.reshape(B, N, OUT_D)


# speedup vs baseline: 1.7958x; 1.0050x over previous
import functools

import jax
import jax.numpy as jnp
from jax import lax
from jax.experimental import pallas as pl
from jax.experimental.pallas import tpu as pltpu
from jax.experimental.pallas import tpu_sc as plsc

B, S, D = 4, 2048, 1024
N = 512
WD = 128
OUT_D = 2 * D + WD
G = B * N
NC, NS, L = 2, 16, 16
NW = NC * NS
SPW = G // NW
CH = 16
NCHUNK = SPW // CH
K = 3

_mesh = plsc.VectorSubcoreMesh(core_axis_name="c", subcore_axis_name="s")


@functools.partial(
    pl.kernel,
    mesh=_mesh,
    out_type=jax.ShapeDtypeStruct((G, OUT_D), jnp.float32),
    scratch_types=(
        [
            pltpu.VMEM((SPW * 2,), jnp.int32),
            pltpu.VMEM((NCHUNK, CH), jnp.int32),
            pltpu.VMEM((NCHUNK, CH), jnp.int32),
            pltpu.VMEM((NCHUNK, CH), jnp.int32),
        ]
        + [pltpu.VMEM((CH, OUT_D), jnp.float32) for _ in range(K)]
        + [pltpu.SemaphoreType.DMA for _ in range(4 * K)]
    ),
)
def _span_extract(seq_hbm, spans_hbm, wtab_hbm, out_hbm,
                  spans_v, sidx_v, eidx_v, widx_v, *bufs):
    orow = bufs[:K]
    gsem = [bufs[K + 3 * k:K + 3 * k + 3] for k in range(K)]
    wsem = bufs[4 * K:5 * K]
    wid = lax.axis_index("s") * NC + lax.axis_index("c")
    base = wid * SPW
    boff = (base // N) * S
    pltpu.sync_copy(spans_hbm.at[pl.ds(base * 2, SPW * 2)], spans_v)
    lane = lax.iota(jnp.int32, L)
    duo = (2 * lane) & (L - 1)
    half = lane < (L // 2)

    def _pick(v, idx):
        return lax.gather(
            v, idx[:, None],
            dimension_numbers=lax.GatherDimensionNumbers(
                offset_dims=(), collapsed_slice_dims=(0,),
                start_index_map=(0,)),
            slice_sizes=(1,),
            mode=lax.GatherScatterMode.PROMISE_IN_BOUNDS)

    for c in range(NCHUNK):
        a = spans_v[pl.ds(c * 2 * L, L)]
        b = spans_v[pl.ds(c * 2 * L + L, L)]
        s = jnp.where(half, _pick(a, duo), _pick(b, duo))
        e = jnp.where(half, _pick(a, duo + 1), _pick(b, duo + 1))
        sidx_v[c] = s + boff
        eidx_v[c] = e + boff
        widx_v[c] = e - s

    gh = [None] * NCHUNK
    wh = [[] for _ in range(K)]
    cols = ((0, D), (D, D), (2 * D, WD))

    def fire(c):
        k = c % K
        for h in wh[k]:
            h.wait()
        wh[k] = []
        gh[c] = tuple(
            pltpu.async_copy(src, orow[k].at[:, pl.ds(lo, w)], gsem[k][i])
            for i, ((lo, w), src) in enumerate(zip(
                cols,
                (seq_hbm.at[sidx_v.at[c]], seq_hbm.at[eidx_v.at[c]],
                 wtab_hbm.at[widx_v.at[c]]))))

    for c in range(min(K, NCHUNK)):
        fire(c)
    for c in range(NCHUNK):
        k = c % K
        rows = pl.ds(base + c * CH, CH)
        for i, (lo, w) in enumerate(cols):
            gh[c][i].wait()
            wh[k].append(pltpu.async_copy(
                orow[k].at[:, pl.ds(lo, w)],
                out_hbm.at[rows, pl.ds(lo, w)], wsem[k]))
        if c + K < NCHUNK:
            fire(c + K)
    for k in range(K):
        for h in wh[k]:
            h.wait()


def kernel(sequence_tensor, span_indices, width_table):
    seq_flat = sequence_tensor.reshape(B * S, D)
    spans_flat = span_indices.astype(jnp.int32).reshape(G * 2)
    out = _span_extract(seq_flat, spans_flat, width_table)
    return out.reshape(B, N, OUT_D)
